# baseline (speedup 1.0000x reference)
import functools

import jax
import jax.numpy as jnp
from jax import lax
from jax.experimental import pallas as pl
from jax.experimental.pallas import tpu as pltpu
from jax.experimental.pallas import tpu_sc as plsc

N, E, D, DA, DE, L, G, DOUT = 10000, 160000, 128, 16, 16, 3, 64, 16
HD = D // 2

NC, NS = 2, 16
NW = NC * NS
EW = E // NW
CH = 40
NCH = EW // CH
WR = 200
NWCH = N // WR
KMAX = -(-NWCH // NS)

NB = 10
BN = N // NB
EBG = 80
BE = E // EBG

_F32 = jnp.float32


def _dot(a, b):
    return jnp.dot(a, b, preferred_element_type=_F32)


def _silu(v):
    return v / (1.0 + jnp.exp(-v))


def _split_store(ref, v):
    ref[0] = v[:, :HD]
    ref[1] = v[:, HD:]


def _merge_halves(g_ref):
    return jnp.concatenate([g_ref[0, 0] + g_ref[1, 0],
                            g_ref[0, 1] + g_ref[1, 1]], axis=1)


def _embed_body(x_ref, na_ref, wex, wea, wms, wmd, h_ref, a_ref, b_ref):
    h = _dot(x_ref[...], wex[...]) * _dot(na_ref[...], wea[...])
    h_ref[...] = h
    _split_store(a_ref, _dot(h, wms[...]))
    _split_store(b_ref, _dot(h, wmd[...]))


def _eproj_body(ea_ref, w0, w1, w2, o0, o1, o2):
    ea = ea_ref[...]
    _split_store(o0, _dot(ea, w0[...]))
    _split_store(o1, _dot(ea, w1[...]))
    _split_store(o2, _dot(ea, w2[...]))


def _update_body(h_ref, g_ref, na_ref, wuh, wug, wua, wms, wmd,
                 ho_ref, ao_ref, bo_ref):
    h = h_ref[...]
    g = _merge_halves(g_ref)
    h2 = h + (_dot(h, wuh[...]) + _dot(g, wug[...])) * _dot(na_ref[...], wua[...])
    ho_ref[...] = h2
    _split_store(ao_ref, _dot(h2, wms[...]))
    _split_store(bo_ref, _dot(h2, wmd[...]))


def _final_body(h_ref, g_ref, na_ref, b_ref, wuh, wug, wua,
                wp0, wpa0, wp1, wpa1, wo1, wo2, out_ref, sums, cnt):
    i = pl.program_id(0)

    @pl.when(i == 0)
    def _():
        sums[...] = jnp.zeros((G, D), _F32)
        cnt[...] = jnp.zeros((G, D), _F32)

    h = h_ref[...]
    na = na_ref[...]
    g = _merge_halves(g_ref)
    h2 = h + (_dot(h, wuh[...]) + _dot(g, wug[...])) * _dot(na, wua[...])
    hp = _silu(_dot(h2, wp0[...]) * _dot(na, wpa0[...]))
    hq = _dot(hp, wp1[...]) * _dot(na, wpa1[...])
    b = b_ref[0, 0, :]
    eqt = (lax.broadcasted_iota(jnp.int32, (G, BN), 0) == b[None, :]).astype(_F32)
    sums[...] += _dot(eqt, hq)
    cnt[...] += jnp.broadcast_to(jnp.sum(eqt, axis=1, keepdims=True), (G, D))

    @pl.when(i == NB - 1)
    def _():
        pooled = sums[...] / jnp.maximum(cnt[...], 1.0)
        z = _silu(_dot(pooled, wo1[...]))
        out_ref[...] = _dot(z, wo2[...])


def _node_spec(i):
    return (i, 0)


_W_SPEC = pl.BlockSpec((D, D), lambda i: (0, 0))
_WA_SPEC = pl.BlockSpec((DA, D), lambda i: (0, 0))
_H_SPEC = pl.BlockSpec((BN, D), _node_spec)
_NA_SPEC = pl.BlockSpec((BN, DA), _node_spec)
_AB_SPEC = pl.BlockSpec((2, BN, HD), lambda i: (0, i, 0))
_G_SPEC = pl.BlockSpec((NC, 2, BN, HD), lambda i: (0, 0, i, 0))
_AB_SHAPE = jax.ShapeDtypeStruct((2, N, HD), _F32)
_EP_SHAPE = jax.ShapeDtypeStruct((2, E, HD), _F32)

_embed_call = pl.pallas_call(
    _embed_body,
    grid=(NB,),
    in_specs=[_H_SPEC, _NA_SPEC, _W_SPEC, _WA_SPEC, _W_SPEC, _W_SPEC],
    out_specs=[_H_SPEC, _AB_SPEC, _AB_SPEC],
    out_shape=[jax.ShapeDtypeStruct((N, D), _F32), _AB_SHAPE, _AB_SHAPE],
)

_eproj_call = pl.pallas_call(
    _eproj_body,
    grid=(EBG,),
    in_specs=[pl.BlockSpec((BE, DE), _node_spec)] + [pl.BlockSpec((DE, D), lambda i: (0, 0))] * 3,
    out_specs=[pl.BlockSpec((2, BE, HD), lambda i: (0, i, 0))] * 3,
    out_shape=[_EP_SHAPE] * 3,
)

_update_call = pl.pallas_call(
    _update_body,
    grid=(NB,),
    in_specs=[_H_SPEC, _G_SPEC, _NA_SPEC, _W_SPEC, _W_SPEC, _WA_SPEC, _W_SPEC, _W_SPEC],
    out_specs=[_H_SPEC, _AB_SPEC, _AB_SPEC],
    out_shape=[jax.ShapeDtypeStruct((N, D), _F32), _AB_SHAPE, _AB_SHAPE],
)

_final_call = pl.pallas_call(
    _final_body,
    grid=(NB,),
    in_specs=[_H_SPEC, _G_SPEC, _NA_SPEC,
              pl.BlockSpec((1, 1, BN), lambda i: (i, 0, 0)),
              _W_SPEC, _W_SPEC, _WA_SPEC,
              _W_SPEC, _WA_SPEC, _W_SPEC, _WA_SPEC,
              _W_SPEC, pl.BlockSpec((D, DOUT), lambda i: (0, 0))],
    out_specs=pl.BlockSpec((G, DOUT), lambda i: (0, 0)),
    out_shape=jax.ShapeDtypeStruct((G, DOUT), _F32),
    scratch_shapes=[pltpu.VMEM((G, D), _F32), pltpu.VMEM((G, D), _F32)],
)


def _edge_body(a_hbm, b_hbm, ep_hbm, src_hbm, dst_hbm, dis_hbm, wd_hbm,
               out_hbm, src_v, dst_v, dis_v, wd_v, a_b, b_b, e_b, m_b,
               stage, agg_sh, sem_a, sem_b, sem_e, sem_d):
    cid = lax.axis_index("c")
    sid = lax.axis_index("s")
    wid = sid * NC + cid

    pltpu.sync_copy(src_hbm.at[wid], src_v)
    pltpu.sync_copy(dst_hbm.at[wid], dst_v)
    pltpu.sync_copy(wd_hbm, wd_v)

    def _zero_row(i, carry):
        for j in range(HD // 16):
            stage[i, pl.ds(16 * j, 16)] = jnp.zeros((16,), _F32)
        return carry

    lax.fori_loop(0, WR, _zero_row, 0)

    ebase = wid * EW

    for p in range(2):
        for k in range(KMAX):
            cc = sid + NS * k

            @pl.when(cc < NWCH)
            def _():
                pltpu.sync_copy(stage, agg_sh.at[pl.ds(cc * WR, WR)])

        plsc.subcore_barrier()

        def _chunk(c, carry):
            da = pltpu.async_copy(a_hbm.at[p].at[src_v.at[c]], a_b, sem_a)
            db = pltpu.async_copy(b_hbm.at[p].at[dst_v.at[c]], b_b, sem_b)
            de = pltpu.async_copy(ep_hbm.at[p, pl.ds(ebase + c * CH, CH)],
                                  e_b, sem_e)
            dd = pltpu.async_copy(dis_hbm.at[wid, c], dis_v, sem_d)
            da.wait()
            db.wait()
            de.wait()
            dd.wait()

            def _edge(i, carry2):
                dvec = dis_v[i, :]
                for j in range(HD // 16):
                    sl = pl.ds(16 * j, 16)
                    w = wd_v[pl.ds(p * HD + 16 * j, 16)]
                    t = (a_b[i, sl] + b_b[i, sl] + dvec * w) * e_b[i, sl]
                    m_b[i, sl] = _silu(t)
                return carry2

            lax.fori_loop(0, CH, _edge, 0)
            pltpu.sync_copy(m_b, agg_sh.at[dst_v.at[c]], add=True)
            return carry

        lax.fori_loop(0, NCH, _chunk, 0)
        plsc.subcore_barrier()

        for k in range(KMAX):
            cc = sid + NS * k

            @pl.when(cc < NWCH)
            def _():
                sl = pl.ds(cc * WR, WR)
                pltpu.sync_copy(agg_sh.at[sl], stage)
                pltpu.sync_copy(stage, out_hbm.at[cid, p, sl])

        if p == 0:
            lax.fori_loop(0, WR, _zero_row, 0)


@functools.lru_cache(maxsize=1)
def _make_edge_call():
  return functools.partial(
    pl.kernel,
    out_type=jax.ShapeDtypeStruct((NC, 2, N, HD), _F32),
    mesh=plsc.VectorSubcoreMesh(core_axis_name="c", subcore_axis_name="s",
                                num_cores=NC, num_subcores=NS),
    compiler_params=pltpu.CompilerParams(use_tc_tiling_on_sc=False),
    scratch_types=[
        pltpu.VMEM((NCH, CH), jnp.int32),
        pltpu.VMEM((NCH, CH), jnp.int32),
        pltpu.VMEM((CH, 16), _F32),
        pltpu.VMEM((D,), _F32),
        pltpu.VMEM((CH, HD), _F32),
        pltpu.VMEM((CH, HD), _F32),
        pltpu.VMEM((CH, HD), _F32),
        pltpu.VMEM((CH, HD), _F32),
        pltpu.VMEM((WR, HD), _F32),
        pltpu.VMEM_SHARED((N, HD), _F32),
        pltpu.SemaphoreType.DMA,
        pltpu.SemaphoreType.DMA,
        pltpu.SemaphoreType.DMA,
        pltpu.SemaphoreType.DMA,
    ],
  )(_edge_body)


def _edge_call(*args):
    return _make_edge_call()(*args)


def kernel(x, edge_index, edge_attr, node_attr, batch, edge_dis,
           Wemb_x, Wemb_a, Wm, Wme, Wu, Wua, Wp, Wpa, Wo1, Wo2):
    src_r = edge_index[0].reshape(NW, NCH, CH)
    dst_r = edge_index[1].reshape(NW, NCH, CH)
    dis_r = jnp.broadcast_to(edge_dis.reshape(NW, NCH, CH, 1), (NW, NCH, CH, 16))
    batch_r = batch.reshape(NB, 1, BN)
    wm_src = Wm[:, :D, :]
    wm_dst = Wm[:, D:2 * D, :]
    wm_dis = Wm[:, 2 * D, :]
    wu_h = Wu[:, :D, :]
    wu_g = Wu[:, D:, :]

    h, a, b = _embed_call(x, node_attr, Wemb_x, Wemb_a, wm_src[0], wm_dst[0])
    ep = _eproj_call(edge_attr, Wme[0], Wme[1], Wme[2])

    for l in range(L - 1):
        agg2 = _edge_call(a, b, ep[l], src_r, dst_r, dis_r, wm_dis[l])
        h, a, b = _update_call(h, agg2, node_attr, wu_h[l], wu_g[l], Wua[l],
                               wm_src[l + 1], wm_dst[l + 1])

    agg2 = _edge_call(a, b, ep[L - 1], src_r, dst_r, dis_r, wm_dis[L - 1])
    out = _final_call(h, agg2, node_attr, batch_r, wu_h[L - 1], wu_g[L - 1],
                      Wua[L - 1], Wp[0], Wpa[0], Wp[1], Wpa[1], Wo1, Wo2)
    return out
